# baseline (speedup 1.0000x reference)
import jax
import jax.numpy as jnp
from jax.experimental import pallas as pl

_N, _D, _H, _K = 10000, 512, 512, 8
_BR = 400
_NPAD = 10240
_BC = 1024
_NCHUNK = _NPAD // _BC
_ENC_BR = 1000

_NEG = -3.0e38
_IDX_SENTINEL = 2**30


def _enc_kernel(x_ref, w_ref, b_ref, y_ref, s_ref):
    y = jnp.dot(x_ref[...], w_ref[...], preferred_element_type=jnp.float32)
    y = jnp.maximum(y + b_ref[...], 0.0)
    y_ref[...] = y
    n = jnp.sqrt(jnp.sum(y * y, axis=1, keepdims=True))
    s_ref[...] = y / jnp.maximum(n, 1e-12)


def _encode(x, w, b):
    return pl.pallas_call(
        _enc_kernel,
        grid=(_N // _ENC_BR,),
        in_specs=[
            pl.BlockSpec((_ENC_BR, _D), lambda i: (i, 0)),
            pl.BlockSpec((_D, _H), lambda i: (0, 0)),
            pl.BlockSpec((1, _H), lambda i: (0, 0)),
        ],
        out_specs=[
            pl.BlockSpec((_ENC_BR, _H), lambda i: (i, 0)),
            pl.BlockSpec((_ENC_BR, _H), lambda i: (i, 0)),
        ],
        out_shape=[
            jax.ShapeDtypeStruct((_N, _H), jnp.float32),
            jax.ShapeDtypeStruct((_N, _H), jnp.float32),
        ],
    )(x, w, b.reshape(1, _H))


def _knn_kernel(s_ref, tT_ref, idx_ref):
    s = s_ref[...]
    li128 = jax.lax.broadcasted_iota(jnp.int32, (_BR, 128), 1)
    nplanes = _BC // 128
    def _dot(c):
        return jnp.dot(s, tT_ref[:, c * _BC:(c + 1) * _BC],
                       preferred_element_type=jnp.float32)
    sims = [_dot(0), _dot(1)]
    RV, RI = None, None
    for c in range(_NCHUNK):
        if c + 2 < _NCHUNK:
            sims.append(_dot(c + 2))
        sim = sims[c]
        V = [sim[:, j * 128:(j + 1) * 128] for j in range(nplanes)]
        I = [li128 + (c * _BC + j * 128) for j in range(nplanes)]
        for j in range(nplanes):
            lim = _N - c * _BC - j * 128
            if lim >= 128:
                continue
            lim = max(lim, 0)
            V[j] = jnp.where(li128 < lim, V[j], _NEG)
        for a, b in ((0, 1), (2, 3), (4, 5), (6, 7),
                     (0, 2), (1, 3), (4, 6), (5, 7),
                     (1, 2), (5, 6),
                     (0, 4), (1, 5), (2, 6), (3, 7),
                     (2, 4), (3, 5),
                     (1, 2), (3, 4), (5, 6)):
            cswap = V[b] > V[a]
            vhi = jnp.maximum(V[a], V[b])
            vlo = jnp.minimum(V[a], V[b])
            ihi = jnp.where(cswap, I[b], I[a])
            ilo = jnp.where(cswap, I[a], I[b])
            V[a], V[b], I[a], I[b] = vhi, vlo, ihi, ilo
        if RV is None:
            RV, RI = V, I
            continue
        MV, MI = [], []
        for i in range(nplanes):
            cs = V[nplanes - 1 - i] > RV[i]
            MV.append(jnp.maximum(RV[i], V[nplanes - 1 - i]))
            MI.append(jnp.where(cs, I[nplanes - 1 - i], RI[i]))
        for d in (4, 2, 1):
            for a in range(nplanes):
                b = a + d
                if b >= nplanes or (a // d) % 2 == 1:
                    continue
                cswap = MV[b] > MV[a]
                vhi = jnp.maximum(MV[a], MV[b])
                vlo = jnp.minimum(MV[a], MV[b])
                ihi = jnp.where(cswap, MI[b], MI[a])
                ilo = jnp.where(cswap, MI[a], MI[b])
                MV[a], MV[b], MI[a], MI[b] = vhi, vlo, ihi, ilo
        RV, RI = MV, MI
    out_i = []
    for t in range(_K):
        m = jnp.max(RV[0], axis=1, keepdims=True)
        cand = jnp.where(RV[0] == m, RI[0], _IDX_SENTINEL)
        sel = jnp.min(cand, axis=1, keepdims=True)
        out_i.append(sel)
        if t == _K - 1:
            break
        f = cand == sel
        depth = nplanes - 1 - t
        for rr in range(depth):
            RV[rr] = jnp.where(f, RV[rr + 1], RV[rr])
            RI[rr] = jnp.where(f, RI[rr + 1], RI[rr])
        RV[depth] = jnp.where(f, _NEG, RV[depth])
    idx_ref[...] = jnp.concatenate(out_i, axis=1)


def _knn(s, t):
    tT = jnp.pad(t, ((0, _NPAD - _N), (0, 0))).T
    return pl.pallas_call(
        _knn_kernel,
        grid=(_N // _BR,),
        in_specs=[
            pl.BlockSpec((_BR, _H), lambda i: (i, 0)),
            pl.BlockSpec((_H, _NPAD), lambda i: (0, 0)),
        ],
        out_specs=pl.BlockSpec((_BR, _K), lambda i: (i, 0)),
        out_shape=jax.ShapeDtypeStruct((_N, _K), jnp.int32),
    )(s, tT)


def kernel(online_x, target_x, edge_index, W_enc, b_enc, W_enc_t, b_enc_t,
           W_p1, b_p1, W_p2, b_p2, k):
    online_y, s = _encode(online_x, W_enc, b_enc)
    target_y, t = _encode(target_x, W_enc_t, b_enc_t)
    I_knn = _knn(s, t)
    rows = jnp.repeat(jnp.arange(_N, dtype=jnp.int32), _K)
    knn = jnp.stack([rows, I_knn.reshape(-1)], axis=0)
    return (online_y, target_y, knn)
